# Initial kernel scaffold; baseline (speedup 1.0000x reference)
import jax
import jax.numpy as jnp
from jax.experimental import pallas as pl


def kernel(x, m1, m2, m3, m4, biases):
    raise NotImplementedError("write your pallas kernel here")


# speedup vs baseline: 1.0061x; 1.0061x over previous
import jax
import jax.numpy as jnp
from jax.experimental import pallas as pl
from jax.experimental.pallas import tpu as pltpu

_IN, _H, _OUT = 4, 8, 3
_G = 128 // _IN
_GH = _G * _H
_GO = _G * _OUT
_ROW_TILE = 4096


def _mlp_body(x_ref, m1_ref, m2_ref, m3_ref, m4_ref, b_ref, o_ref):
    x = x_ref[...]
    b1 = b_ref[0:1, :]
    b2 = b_ref[1:2, :]
    b3 = b_ref[2:3, :]
    b4 = b_ref[3:4, 0:_GO]
    h = jnp.dot(x, m1_ref[...], preferred_element_type=jnp.float32) + b1
    h = jnp.maximum(h, 0.0)
    h = jnp.dot(h, m2_ref[...], preferred_element_type=jnp.float32) + b2
    h = jnp.maximum(h, 0.0)
    h = jnp.dot(h, m3_ref[...], preferred_element_type=jnp.float32) + b3
    h = jnp.maximum(h, 0.0)
    out = jnp.dot(h, m4_ref[...], preferred_element_type=jnp.float32)
    o_ref[...] = out[:, :_GO] + b4


def kernel(x, m1, m2, m3, m4, biases):
    B = x.shape[0]
    x = jnp.asarray(x, jnp.float32)

    r_total = pl.cdiv(B, _G)
    n_blocks = pl.cdiv(r_total, _ROW_TILE)
    rt = _ROW_TILE if n_blocks > 1 else max(8, ((r_total + 7) // 8) * 8)
    n_blocks = pl.cdiv(r_total, rt)
    r_pad = rt * n_blocks
    b_pad = r_pad * _G
    if b_pad != B:
        x = jnp.pad(x, ((0, b_pad - B), (0, 0)))
    x_rows = x.reshape(r_pad, _G * _IN)

    m4p = jnp.pad(jnp.asarray(m4, jnp.float32), ((0, 0), (0, _GH - _GO)))

    out_rows = pl.pallas_call(
        _mlp_body,
        out_shape=jax.ShapeDtypeStruct((r_pad, _GO), jnp.float32),
        grid_spec=pl.GridSpec(
            grid=(n_blocks,),
            in_specs=[
                pl.BlockSpec((rt, _G * _IN), lambda i: (i, 0)),
                pl.BlockSpec((_G * _IN, _GH), lambda i: (0, 0)),
                pl.BlockSpec((_GH, _GH), lambda i: (0, 0)),
                pl.BlockSpec((_GH, _GH), lambda i: (0, 0)),
                pl.BlockSpec((_GH, _GH), lambda i: (0, 0)),
                pl.BlockSpec((4, _GH), lambda i: (0, 0)),
            ],
            out_specs=pl.BlockSpec((rt, _GO), lambda i: (i, 0)),
        ),
        compiler_params=pltpu.CompilerParams(
            dimension_semantics=("parallel",),
            vmem_limit_bytes=64 * 1024 * 1024,
        ),
    )(x_rows, jnp.asarray(m1, jnp.float32), jnp.asarray(m2, jnp.float32),
      jnp.asarray(m3, jnp.float32), m4p, jnp.asarray(biases, jnp.float32))

    out = out_rows.reshape(b_pad, _OUT)
    return out[:B] if b_pad != B else out


# speedup vs baseline: 10.2629x; 10.2005x over previous
import jax
import jax.numpy as jnp
from jax.experimental import pallas as pl
from jax.experimental.pallas import tpu as pltpu

_IN, _H, _OUT = 4, 8, 3
_N_BLOCKS = 16


def _mlp_t_body(xt_ref, m1_ref, m2_ref, m3_ref, m4t_ref, b_ref, o_ref):
    c1 = jnp.transpose(m1_ref[0:_IN, 0:_H])
    c2 = jnp.transpose(m2_ref[0:_H, 0:_H])
    c3 = jnp.transpose(m3_ref[0:_H, 0:_H])
    c4 = m4t_ref[0:_OUT, 0:_H]
    bt = jnp.transpose(b_ref[0:4, 0:_H])

    x = xt_ref[...]

    h = bt[:, 0:1] + sum(c1[:, k:k + 1] * x[k:k + 1, :] for k in range(_IN))
    h = jnp.maximum(h, 0.0)
    h = bt[:, 1:2] + sum(c2[:, k:k + 1] * h[k:k + 1, :] for k in range(_H))
    h = jnp.maximum(h, 0.0)
    h = bt[:, 2:3] + sum(c3[:, k:k + 1] * h[k:k + 1, :] for k in range(_H))
    h = jnp.maximum(h, 0.0)
    out = bt[0:_OUT, 3:4] + sum(
        c4[:, k:k + 1] * h[k:k + 1, :] for k in range(_H))
    o_ref[...] = out


def kernel(x, m1, m2, m3, m4, biases):
    B = x.shape[0]
    xt = jnp.swapaxes(jnp.asarray(x, jnp.float32), 0, 1)

    lt = pl.cdiv(B, _N_BLOCKS)
    lt = ((lt + 127) // 128) * 128
    n_blocks = pl.cdiv(B, lt)
    b_pad = lt * n_blocks
    if b_pad != B:
        xt = jnp.pad(xt, ((0, 0), (0, b_pad - B)))

    out_t = pl.pallas_call(
        _mlp_t_body,
        out_shape=jax.ShapeDtypeStruct((_OUT, b_pad), jnp.float32),
        grid_spec=pl.GridSpec(
            grid=(n_blocks,),
            in_specs=[
                pl.BlockSpec((_IN, lt), lambda i: (0, i)),
                pl.BlockSpec(m1.shape, lambda i: (0, 0)),
                pl.BlockSpec(m2.shape, lambda i: (0, 0)),
                pl.BlockSpec(m3.shape, lambda i: (0, 0)),
                pl.BlockSpec((m4.shape[1], m4.shape[0]), lambda i: (0, 0)),
                pl.BlockSpec(biases.shape, lambda i: (0, 0)),
            ],
            out_specs=pl.BlockSpec((_OUT, lt), lambda i: (0, i)),
        ),
        compiler_params=pltpu.CompilerParams(
            dimension_semantics=("parallel",),
            vmem_limit_bytes=64 * 1024 * 1024,
        ),
    )(xt, jnp.asarray(m1, jnp.float32), jnp.asarray(m2, jnp.float32),
      jnp.asarray(m3, jnp.float32),
      jnp.swapaxes(jnp.asarray(m4, jnp.float32), 0, 1),
      jnp.asarray(biases, jnp.float32))

    out = jnp.swapaxes(out_t, 0, 1)
    return out[:B] if b_pad != B else out


# speedup vs baseline: 85.2865x; 8.3102x over previous
import jax
import jax.numpy as jnp
from jax.experimental import pallas as pl
from jax.experimental.pallas import tpu as pltpu

_IN, _H, _OUT = 4, 8, 3
_N_BLOCKS = 32


def _mlp_t_body(xt_ref, m1_ref, m2_ref, m3_ref, m4t_ref, b_ref, o_ref):
    c1 = jnp.transpose(m1_ref[0:_IN, 0:_H])
    c2 = jnp.transpose(m2_ref[0:_H, 0:_H])
    c3 = jnp.transpose(m3_ref[0:_H, 0:_H])
    c4 = m4t_ref[0:_OUT, 0:_H]
    bt = jnp.transpose(b_ref[0:4, 0:_H])

    x = xt_ref[...]

    f32 = jnp.float32
    h = jnp.dot(c1, x, preferred_element_type=f32) + bt[:, 0:1]
    h = jnp.maximum(h, 0.0)
    h = jnp.dot(c2, h, preferred_element_type=f32) + bt[:, 1:2]
    h = jnp.maximum(h, 0.0)
    h = jnp.dot(c3, h, preferred_element_type=f32) + bt[:, 2:3]
    h = jnp.maximum(h, 0.0)
    o_ref[...] = jnp.dot(c4, h, preferred_element_type=f32) + bt[0:_OUT, 3:4]


def kernel(x, m1, m2, m3, m4, biases):
    B = x.shape[0]
    xt = jnp.swapaxes(jnp.asarray(x, jnp.float32), 0, 1)

    lt = pl.cdiv(B, _N_BLOCKS)
    lt = ((lt + 127) // 128) * 128
    n_blocks = pl.cdiv(B, lt)
    b_pad = lt * n_blocks
    if b_pad != B:
        xt = jnp.pad(xt, ((0, 0), (0, b_pad - B)))

    out_t = pl.pallas_call(
        _mlp_t_body,
        out_shape=jax.ShapeDtypeStruct((_OUT, b_pad), jnp.float32),
        grid_spec=pl.GridSpec(
            grid=(n_blocks,),
            in_specs=[
                pl.BlockSpec((_IN, lt), lambda i: (0, i)),
                pl.BlockSpec(m1.shape, lambda i: (0, 0)),
                pl.BlockSpec(m2.shape, lambda i: (0, 0)),
                pl.BlockSpec(m3.shape, lambda i: (0, 0)),
                pl.BlockSpec((m4.shape[1], m4.shape[0]), lambda i: (0, 0)),
                pl.BlockSpec(biases.shape, lambda i: (0, 0)),
            ],
            out_specs=pl.BlockSpec((_OUT, lt), lambda i: (0, i)),
        ),
        compiler_params=pltpu.CompilerParams(
            dimension_semantics=("parallel",),
            vmem_limit_bytes=64 * 1024 * 1024,
        ),
    )(xt, jnp.asarray(m1, jnp.float32), jnp.asarray(m2, jnp.float32),
      jnp.asarray(m3, jnp.float32),
      jnp.swapaxes(jnp.asarray(m4, jnp.float32), 0, 1),
      jnp.asarray(biases, jnp.float32))

    out = jnp.swapaxes(out_t, 0, 1)
    return out[:B] if b_pad != B else out


# speedup vs baseline: 102.6473x; 1.2036x over previous
import jax
import jax.numpy as jnp
from jax.experimental import pallas as pl
from jax.experimental.pallas import tpu as pltpu

_IN, _H, _OUT = 4, 8, 3
_N_BLOCKS = 16


def _mlp_t_body(xt_ref, m1_ref, m2_ref, m3_ref, m4t_ref, b_ref, o_ref):
    c1 = jnp.transpose(m1_ref[0:_IN, 0:_H])
    c2 = jnp.transpose(m2_ref[0:_H, 0:_H])
    c3 = jnp.transpose(m3_ref[0:_H, 0:_H])
    c4 = m4t_ref[0:_OUT, 0:_H]
    bt = jnp.transpose(b_ref[0:4, 0:_H])

    x = xt_ref[...]

    f32 = jnp.float32
    h = jnp.dot(c1, x, preferred_element_type=f32) + bt[:, 0:1]
    h = jnp.maximum(h, 0.0)
    h = jnp.dot(c2, h, preferred_element_type=f32) + bt[:, 1:2]
    h = jnp.maximum(h, 0.0)
    h = jnp.dot(c3, h, preferred_element_type=f32) + bt[:, 2:3]
    h = jnp.maximum(h, 0.0)
    o_ref[...] = jnp.dot(c4, h, preferred_element_type=f32) + bt[0:_OUT, 3:4]


def kernel(x, m1, m2, m3, m4, biases):
    B = x.shape[0]
    xt = jnp.swapaxes(jnp.asarray(x, jnp.float32), 0, 1)

    lt = pl.cdiv(B, _N_BLOCKS)
    lt = ((lt + 127) // 128) * 128
    n_blocks = pl.cdiv(B, lt)
    b_pad = lt * n_blocks
    if b_pad != B:
        xt = jnp.pad(xt, ((0, 0), (0, b_pad - B)))

    out_t = pl.pallas_call(
        _mlp_t_body,
        out_shape=jax.ShapeDtypeStruct((_OUT, b_pad), jnp.float32),
        grid_spec=pl.GridSpec(
            grid=(n_blocks,),
            in_specs=[
                pl.BlockSpec((_IN, lt), lambda i: (0, i)),
                pl.BlockSpec(m1.shape, lambda i: (0, 0)),
                pl.BlockSpec(m2.shape, lambda i: (0, 0)),
                pl.BlockSpec(m3.shape, lambda i: (0, 0)),
                pl.BlockSpec((m4.shape[1], m4.shape[0]), lambda i: (0, 0)),
                pl.BlockSpec(biases.shape, lambda i: (0, 0)),
            ],
            out_specs=pl.BlockSpec((_OUT, lt), lambda i: (0, i)),
        ),
        compiler_params=pltpu.CompilerParams(
            dimension_semantics=("parallel",),
            vmem_limit_bytes=64 * 1024 * 1024,
        ),
    )(xt, jnp.asarray(m1, jnp.float32), jnp.asarray(m2, jnp.float32),
      jnp.asarray(m3, jnp.float32),
      jnp.swapaxes(jnp.asarray(m4, jnp.float32), 0, 1),
      jnp.asarray(biases, jnp.float32))

    out = jnp.swapaxes(out_t, 0, 1)
    return out[:B] if b_pad != B else out


# speedup vs baseline: 105.1788x; 1.0247x over previous
import jax
import jax.numpy as jnp
from jax.experimental import pallas as pl
from jax.experimental.pallas import tpu as pltpu

_IN, _H, _OUT = 4, 8, 3
_N_BLOCKS = 8


def _mlp_t_body(xt_ref, m1_ref, m2_ref, m3_ref, m4t_ref, b_ref, o_ref):
    c1 = jnp.transpose(m1_ref[0:_IN, 0:_H])
    c2 = jnp.transpose(m2_ref[0:_H, 0:_H])
    c3 = jnp.transpose(m3_ref[0:_H, 0:_H])
    c4 = m4t_ref[0:_OUT, 0:_H]
    bt = jnp.transpose(b_ref[0:4, 0:_H])

    x = xt_ref[...]

    f32 = jnp.float32
    h = jnp.dot(c1, x, preferred_element_type=f32) + bt[:, 0:1]
    h = jnp.maximum(h, 0.0)
    h = jnp.dot(c2, h, preferred_element_type=f32) + bt[:, 1:2]
    h = jnp.maximum(h, 0.0)
    h = jnp.dot(c3, h, preferred_element_type=f32) + bt[:, 2:3]
    h = jnp.maximum(h, 0.0)
    o_ref[...] = jnp.dot(c4, h, preferred_element_type=f32) + bt[0:_OUT, 3:4]


def kernel(x, m1, m2, m3, m4, biases):
    B = x.shape[0]
    xt = jnp.swapaxes(jnp.asarray(x, jnp.float32), 0, 1)

    lt = pl.cdiv(B, _N_BLOCKS)
    lt = ((lt + 127) // 128) * 128
    n_blocks = pl.cdiv(B, lt)
    b_pad = lt * n_blocks
    if b_pad != B:
        xt = jnp.pad(xt, ((0, 0), (0, b_pad - B)))

    out_t = pl.pallas_call(
        _mlp_t_body,
        out_shape=jax.ShapeDtypeStruct((_OUT, b_pad), jnp.float32),
        grid_spec=pl.GridSpec(
            grid=(n_blocks,),
            in_specs=[
                pl.BlockSpec((_IN, lt), lambda i: (0, i)),
                pl.BlockSpec(m1.shape, lambda i: (0, 0)),
                pl.BlockSpec(m2.shape, lambda i: (0, 0)),
                pl.BlockSpec(m3.shape, lambda i: (0, 0)),
                pl.BlockSpec((m4.shape[1], m4.shape[0]), lambda i: (0, 0)),
                pl.BlockSpec(biases.shape, lambda i: (0, 0)),
            ],
            out_specs=pl.BlockSpec((_OUT, lt), lambda i: (0, i)),
        ),
        compiler_params=pltpu.CompilerParams(
            dimension_semantics=("parallel",),
            vmem_limit_bytes=64 * 1024 * 1024,
        ),
    )(xt, jnp.asarray(m1, jnp.float32), jnp.asarray(m2, jnp.float32),
      jnp.asarray(m3, jnp.float32),
      jnp.swapaxes(jnp.asarray(m4, jnp.float32), 0, 1),
      jnp.asarray(biases, jnp.float32))

    out = jnp.swapaxes(out_t, 0, 1)
    return out[:B] if b_pad != B else out
